# baseline (speedup 1.0000x reference)
import jax
import jax.numpy as jnp
from jax import lax
from jax.experimental import pallas as pl
from jax.experimental.pallas import tpu as pltpu
from jax.experimental.pallas import tpu_sc as plsc

NC = 2
NS = 16
NW = NC * NS
CH = 128
VB = 128


def _transpose_body(tT, tail_pairs, out, f0, f1, v0, v1, si0, si1, so0, so1):
    nf = tT.shape[0]
    nv = tT.shape[1]
    nblk = nv // VB
    c = lax.axis_index("c")
    s = lax.axis_index("s")
    wid = s * NC + c
    per_w = -(-nblk // NW)
    if per_w % 2:
        per_w += 1

    iota16 = lax.iota(jnp.int32, 16)
    fidx = [iota16 + 16 * q for q in range(nf // 16)]

    def m_of(t):
        m = wid + NW * t
        return jnp.where(m < nblk, m, wid)

    def start_in(t, fbuf, sem):
        return pltpu.async_copy(
            tT.at[:, pl.ds(m_of(t) * VB, VB)], fbuf, sem)

    def start_out(t, vbuf, sem):
        return pltpu.async_copy(
            vbuf, out.at[pl.ds(m_of(t) * (VB // 2), VB // 2)], sem)

    def xpose(fbuf, vbuf, rows):
        for w in range(rows):
            for p in range(2):
                for q in range(nf // 16):
                    col = plsc.load_gather(
                        fbuf, [fidx[q], jnp.full((16,), 2 * w + p, jnp.int32)])
                    vbuf[w, pl.ds(p * nf + 16 * q, 16)] = col

    def drain_in(fbuf, sem):
        pltpu.make_async_copy(tT.at[:, pl.ds(0, VB)], fbuf, sem).wait()

    def drain_out(vbuf, sem):
        pltpu.make_async_copy(
            vbuf, out.at[pl.ds(0, VB // 2)], sem).wait()

    start_in(0, f0, si0)
    start_in(1, f1, si1)
    drain_in(f0, si0)
    xpose(f0, v0, VB // 2)
    start_out(0, v0, so0)
    start_in(2, f0, si0)
    drain_in(f1, si1)
    xpose(f1, v1, VB // 2)
    start_out(1, v1, so1)
    start_in(3, f1, si1)

    def pair(k, carry):
        t0 = 2 * k
        drain_in(f0, si0)
        drain_out(v0, so0)
        xpose(f0, v0, VB // 2)
        start_out(t0, v0, so0)
        start_in(t0 + 2, f0, si0)
        drain_in(f1, si1)
        drain_out(v1, so1)
        xpose(f1, v1, VB // 2)
        start_out(t0 + 1, v1, so1)
        start_in(t0 + 3, f1, si1)
        return carry

    lax.fori_loop(1, per_w // 2, pair, 0)

    drain_in(f0, si0)
    drain_in(f1, si1)
    drain_out(v0, so0)
    drain_out(v1, so1)

    tail = nv - nblk * VB
    if tail:
        @pl.when(wid == 0)
        def _():
            pltpu.sync_copy(tail_pairs, v0.at[pl.ds(0, tail // 2)])
            pltpu.sync_copy(
                v0.at[pl.ds(0, tail // 2)],
                out.at[pl.ds(nblk * (VB // 2), tail // 2)])


def _sc_body(table, rt, hids, tids, hpar, tpar, rels, out,
             rt_v, hidx, tidx, hp_v, tp_v, rel_v,
             h0, h1, t0, t1, out_v, tmp, sem0, sem1):
    nch = hids.shape[1]
    c = lax.axis_index("c")
    s = lax.axis_index("s")
    wid = s * NC + c

    pltpu.sync_copy(rt, rt_v)
    pltpu.sync_copy(hids.at[wid], hidx)
    pltpu.sync_copy(tids.at[wid], tidx)
    pltpu.sync_copy(hpar.at[wid], hp_v)
    pltpu.sync_copy(tpar.at[wid], tp_v)
    pltpu.sync_copy(rels.at[wid], rel_v)

    r0 = [rt_v[0, pl.ds(16 * q, 16)] for q in range(4)]
    rd = [rt_v[1, pl.ds(16 * q, 16)] - r0[q] for q in range(4)]

    hb = (h0, h1)
    tb = (t0, t1)
    sems = (sem0, sem1)

    def start(j, slot):
        ch = pltpu.async_copy(table.at[hidx.at[j]], hb[slot], sems[slot])
        ct = pltpu.async_copy(table.at[tidx.at[j]], tb[slot], sems[slot])
        return ch, ct

    iota16 = lax.iota(jnp.int32, 16)
    pend = start(0, 0)
    for j in range(nch):
        slot = j & 1
        cur = pend
        if j + 1 < nch:
            pend = start(j + 1, slot ^ 1)
        cur[0].wait()
        cur[1].wait()
        base = j * CH

        def group(g, _, slot=slot, base=base):
            gb = g * 16
            rv = rel_v[pl.ds(base + gb, 16)]
            hpv = hp_v[pl.ds(base + gb, 16)]
            tpv = tp_v[pl.ds(base + gb, 16)]
            for u in range(16):
                i = gb + u
                relf = rv[u]
                ho = hpv[u]
                to = tpv[u]
                acc = None
                for q in range(4):
                    h = hb[slot][i, pl.ds(ho + 16 * q, 16)]
                    t = tb[slot][i, pl.ds(to + 16 * q, 16)]
                    e = (h - t) + (r0[q] + relf * rd[q])
                    acc = e * e if acc is None else acc + e * e
                plsc.store_scatter(
                    tmp, [iota16, jnp.full((16,), u, jnp.int32)], acc)
            tot = None
            for l in range(16):
                row = tmp[l]
                tot = row if tot is None else tot + row
            out_v[pl.ds(base + gb, 16)] = -tot
            return 0

        lax.fori_loop(0, CH // 16, group, 0)

    pltpu.sync_copy(out_v, out.at[wid])


def kernel(itemEmbedding, r_table, head_ids, tail_ids, relation_ids):
    b, k = head_ids.shape
    tot = b * k
    epw = tot // NW
    nch = epw // CH
    n, f = itemEmbedding.shape
    f2 = 2 * f

    hv = head_ids.astype(jnp.int32).reshape(-1)
    tv = tail_ids.astype(jnp.int32).reshape(-1)
    h = (hv >> 1).reshape(NW, nch, CH)
    t = (tv >> 1).reshape(NW, nch, CH)
    hp = ((hv & 1) * f).reshape(NW, epw)
    tp = ((tv & 1) * f).reshape(NW, epw)
    r = relation_ids.astype(jnp.float32).reshape(NW, epw)

    mesh = plsc.VectorSubcoreMesh(core_axis_name="c", subcore_axis_name="s")
    params = pltpu.CompilerParams(needs_layout_passes=False)

    transpose = pl.kernel(
        _transpose_body,
        out_type=jax.ShapeDtypeStruct((n // 2, f2), jnp.float32),
        mesh=mesh,
        scratch_types=[
            pltpu.VMEM((f, VB), jnp.float32),
            pltpu.VMEM((f, VB), jnp.float32),
            pltpu.VMEM((VB // 2, f2), jnp.float32),
            pltpu.VMEM((VB // 2, f2), jnp.float32),
            pltpu.SemaphoreType.DMA,
            pltpu.SemaphoreType.DMA,
            pltpu.SemaphoreType.DMA,
            pltpu.SemaphoreType.DMA,
        ],
        compiler_params=params,
    )
    nblk = n // VB
    tail = n - nblk * VB
    tail_pairs = itemEmbedding[n - tail:].reshape(tail // 2, f2)
    table_pairs = transpose(itemEmbedding.T, tail_pairs)

    run = pl.kernel(
        _sc_body,
        out_type=jax.ShapeDtypeStruct((NW, epw), jnp.float32),
        mesh=mesh,
        scratch_types=[
            pltpu.VMEM((2, f), jnp.float32),
            pltpu.VMEM((nch, CH), jnp.int32),
            pltpu.VMEM((nch, CH), jnp.int32),
            pltpu.VMEM((epw,), jnp.int32),
            pltpu.VMEM((epw,), jnp.int32),
            pltpu.VMEM((epw,), jnp.float32),
            pltpu.VMEM((CH, f2), jnp.float32),
            pltpu.VMEM((CH, f2), jnp.float32),
            pltpu.VMEM((CH, f2), jnp.float32),
            pltpu.VMEM((CH, f2), jnp.float32),
            pltpu.VMEM((epw,), jnp.float32),
            pltpu.VMEM((16, 16), jnp.float32),
            pltpu.SemaphoreType.DMA,
            pltpu.SemaphoreType.DMA,
        ],
        compiler_params=params,
    )
    out = run(table_pairs, r_table, h, t, hp, tp, r)
    return out.reshape(b, k)


# speedup vs baseline: 1.4186x; 1.4186x over previous
import jax
import jax.numpy as jnp
from jax import lax
from jax.experimental import pallas as pl
from jax.experimental.pallas import tpu as pltpu
from jax.experimental.pallas import tpu_sc as plsc

NC = 2
NS = 16
NW = NC * NS
CH = 128
VB = 256


def _transpose_body(tT, tail_pairs, out, f0, f1, v0, v1, si0, si1, so0, so1):
    nf = tT.shape[0]
    nv = tT.shape[1]
    ntile = (nv // 128) * 128
    nblk = ntile // VB
    c = lax.axis_index("c")
    s = lax.axis_index("s")
    wid = s * NC + c
    per_w = -(-nblk // NW)
    if per_w % 2:
        per_w += 1

    iota16 = lax.iota(jnp.int32, 16)
    fidx = [iota16 + 16 * q for q in range(nf // 16)]
    zero16 = jnp.zeros((16,), jnp.int32)
    one16 = jnp.full((16,), 1, jnp.int32)
    two16 = jnp.full((16,), 2, jnp.int32)

    def m_of(t):
        m = wid + NW * t
        return jnp.where(m < nblk, m, wid)

    def start_in(t, fbuf, sem):
        return pltpu.async_copy(
            tT.at[:, pl.ds(m_of(t) * VB, VB)], fbuf, sem)

    def start_out(t, vbuf, sem):
        return pltpu.async_copy(
            vbuf, out.at[pl.ds(m_of(t) * (VB // 2), VB // 2)], sem)

    def xpose(fbuf, vbuf, rows, unroll=8):
        def wbody(tw, se):
            for u in range(unroll):
                w = tw * unroll + u
                so = se + one16
                cols = [plsc.load_gather(fbuf, [fidx[q], se])
                        for q in range(nf // 16)]
                cols += [plsc.load_gather(fbuf, [fidx[q], so])
                         for q in range(nf // 16)]
                for j, col in enumerate(cols):
                    vbuf[w, pl.ds(16 * j, 16)] = col
                se = se + two16
            return se
        lax.fori_loop(0, rows // unroll, wbody, zero16)

    def drain_in(fbuf, sem):
        pltpu.make_async_copy(tT.at[:, pl.ds(0, VB)], fbuf, sem).wait()

    def drain_out(vbuf, sem):
        pltpu.make_async_copy(
            vbuf, out.at[pl.ds(0, VB // 2)], sem).wait()

    start_in(0, f0, si0)
    start_in(1, f1, si1)
    drain_in(f0, si0)
    xpose(f0, v0, VB // 2)
    start_out(0, v0, so0)
    start_in(2, f0, si0)
    drain_in(f1, si1)
    xpose(f1, v1, VB // 2)
    start_out(1, v1, so1)
    start_in(3, f1, si1)

    def pair(k, carry):
        t0 = 2 * k
        drain_in(f0, si0)
        drain_out(v0, so0)
        xpose(f0, v0, VB // 2)
        start_out(t0, v0, so0)
        start_in(t0 + 2, f0, si0)
        drain_in(f1, si1)
        drain_out(v1, so1)
        xpose(f1, v1, VB // 2)
        start_out(t0 + 1, v1, so1)
        start_in(t0 + 3, f1, si1)
        return carry

    lax.fori_loop(1, per_w // 2, pair, 0)

    drain_in(f0, si0)
    drain_in(f1, si1)
    drain_out(v0, so0)
    drain_out(v1, so1)

    tail = nv - nblk * VB
    if tail:
        @pl.when(wid == 0)
        def _():
            pltpu.sync_copy(tail_pairs, v0.at[pl.ds(0, tail // 2)])
            pltpu.sync_copy(
                v0.at[pl.ds(0, tail // 2)],
                out.at[pl.ds(nblk * (VB // 2), tail // 2)])


def _sc_body(table, rt, hids, tids, hpar, tpar, rels, out,
             rt_v, hidx, tidx, hp_v, tp_v, rel_v,
             h0, h1, t0, t1, out_v, tmp, sem0, sem1):
    nch = hids.shape[1]
    c = lax.axis_index("c")
    s = lax.axis_index("s")
    wid = s * NC + c

    pltpu.sync_copy(rt, rt_v)
    pltpu.sync_copy(hids.at[wid], hidx)
    pltpu.sync_copy(tids.at[wid], tidx)
    pltpu.sync_copy(hpar.at[wid], hp_v)
    pltpu.sync_copy(tpar.at[wid], tp_v)
    pltpu.sync_copy(rels.at[wid], rel_v)

    r0 = [rt_v[0, pl.ds(16 * q, 16)] for q in range(4)]
    rd = [rt_v[1, pl.ds(16 * q, 16)] - r0[q] for q in range(4)]

    hb = (h0, h1)
    tb = (t0, t1)
    sems = (sem0, sem1)

    def start(j, slot):
        ch = pltpu.async_copy(table.at[hidx.at[j]], hb[slot], sems[slot])
        ct = pltpu.async_copy(table.at[tidx.at[j]], tb[slot], sems[slot])
        return ch, ct

    iota16 = lax.iota(jnp.int32, 16)
    pend = start(0, 0)
    for j in range(nch):
        slot = j & 1
        cur = pend
        if j + 1 < nch:
            pend = start(j + 1, slot ^ 1)
        cur[0].wait()
        cur[1].wait()
        base = j * CH

        def group(g, _, slot=slot, base=base):
            gb = g * 16
            rv = rel_v[pl.ds(base + gb, 16)]
            hpv = hp_v[pl.ds(base + gb, 16)]
            tpv = tp_v[pl.ds(base + gb, 16)]
            for u in range(16):
                i = gb + u
                relf = rv[u]
                ho = hpv[u]
                to = tpv[u]
                acc = None
                for q in range(4):
                    h = hb[slot][i, pl.ds(ho + 16 * q, 16)]
                    t = tb[slot][i, pl.ds(to + 16 * q, 16)]
                    e = (h - t) + (r0[q] + relf * rd[q])
                    acc = e * e if acc is None else acc + e * e
                plsc.store_scatter(
                    tmp, [iota16, jnp.full((16,), u, jnp.int32)], acc)
            tot = None
            for l in range(16):
                row = tmp[l]
                tot = row if tot is None else tot + row
            out_v[pl.ds(base + gb, 16)] = -tot
            return 0

        lax.fori_loop(0, CH // 16, group, 0)

    pltpu.sync_copy(out_v, out.at[wid])


def kernel(itemEmbedding, r_table, head_ids, tail_ids, relation_ids):
    b, k = head_ids.shape
    tot = b * k
    epw = tot // NW
    nch = epw // CH
    n, f = itemEmbedding.shape
    f2 = 2 * f

    hv = head_ids.astype(jnp.int32).reshape(-1)
    tv = tail_ids.astype(jnp.int32).reshape(-1)
    h = (hv >> 1).reshape(NW, nch, CH)
    t = (tv >> 1).reshape(NW, nch, CH)
    hp = ((hv & 1) * f).reshape(NW, epw)
    tp = ((tv & 1) * f).reshape(NW, epw)
    r = relation_ids.astype(jnp.float32).reshape(NW, epw)

    mesh = plsc.VectorSubcoreMesh(core_axis_name="c", subcore_axis_name="s")
    params = pltpu.CompilerParams(needs_layout_passes=False)

    transpose = pl.kernel(
        _transpose_body,
        out_type=jax.ShapeDtypeStruct((n // 2, f2), jnp.float32),
        mesh=mesh,
        scratch_types=[
            pltpu.VMEM((f, VB), jnp.float32),
            pltpu.VMEM((f, VB), jnp.float32),
            pltpu.VMEM((VB // 2, f2), jnp.float32),
            pltpu.VMEM((VB // 2, f2), jnp.float32),
            pltpu.SemaphoreType.DMA,
            pltpu.SemaphoreType.DMA,
            pltpu.SemaphoreType.DMA,
            pltpu.SemaphoreType.DMA,
        ],
        compiler_params=params,
    )
    tail = n - (n // 128) * 128
    tail_pairs = itemEmbedding[n - tail:].reshape(tail // 2, f2)
    table_pairs = transpose(itemEmbedding.T, tail_pairs)

    run = pl.kernel(
        _sc_body,
        out_type=jax.ShapeDtypeStruct((NW, epw), jnp.float32),
        mesh=mesh,
        scratch_types=[
            pltpu.VMEM((2, f), jnp.float32),
            pltpu.VMEM((nch, CH), jnp.int32),
            pltpu.VMEM((nch, CH), jnp.int32),
            pltpu.VMEM((epw,), jnp.int32),
            pltpu.VMEM((epw,), jnp.int32),
            pltpu.VMEM((epw,), jnp.float32),
            pltpu.VMEM((CH, f2), jnp.float32),
            pltpu.VMEM((CH, f2), jnp.float32),
            pltpu.VMEM((CH, f2), jnp.float32),
            pltpu.VMEM((CH, f2), jnp.float32),
            pltpu.VMEM((epw,), jnp.float32),
            pltpu.VMEM((16, 16), jnp.float32),
            pltpu.SemaphoreType.DMA,
            pltpu.SemaphoreType.DMA,
        ],
        compiler_params=params,
    )
    out = run(table_pairs, r_table, h, t, hp, tp, r)
    return out.reshape(b, k)
